# baseline (speedup 1.0000x reference)
import jax
import jax.numpy as jnp
from jax.experimental import pallas as pl
from jax.experimental.pallas import tpu as pltpu

_NUM_CLASSES = 1000


def _linear_body(x_ref, w_ref, b_ref, o_ref, wb_ref):
    k = pl.program_id(1)

    @pl.when(k == 0)
    def _():
        wb_ref[...] = w_ref[...].astype(jnp.bfloat16)

    xb = x_ref[...].astype(jnp.bfloat16)
    acc = jnp.dot(xb, wb_ref[...], preferred_element_type=jnp.float32)
    o_ref[...] = acc + b_ref[...]


def kernel(x, wp, bp):
    B, D = x.shape
    D_pad, C_pad = wp.shape
    C = _NUM_CLASSES

    tm = 256
    nb_inner = B // tm // 2
    grid = (2, nb_inner)

    cost = pl.CostEstimate(
        flops=2 * B * D_pad * C_pad,
        transcendentals=0,
        bytes_accessed=B * D * 4 + 2 * D_pad * C_pad * 4 + B * C * 4,
    )

    out = pl.pallas_call(
        _linear_body,
        out_shape=jax.ShapeDtypeStruct((B, C_pad), jnp.float32),
        grid=grid,
        in_specs=[
            pl.BlockSpec((tm, D), lambda c, k: (c * nb_inner + k, 0)),
            pl.BlockSpec((D_pad, C_pad), lambda c, k: (0, 0)),
            pl.BlockSpec((1, C_pad), lambda c, k: (0, 0)),
        ],
        out_specs=pl.BlockSpec((tm, C_pad), lambda c, k: (c * nb_inner + k, 0)),
        scratch_shapes=[pltpu.VMEM((D_pad, C_pad), jnp.bfloat16)],
        compiler_params=pltpu.CompilerParams(
            dimension_semantics=("parallel", "arbitrary"),
            vmem_limit_bytes=56 * 1024 * 1024,
        ),
        cost_estimate=cost,
    )(x, wp, bp)
    return out[:, :C]


# speedup vs baseline: 1.3946x; 1.3946x over previous
import jax
import jax.numpy as jnp
from jax.experimental import pallas as pl
from jax.experimental.pallas import tpu as pltpu

_NUM_CLASSES = 1000


def _linear_body(x_ref, w_ref, b_ref, o_ref, wb_ref):
    k = pl.program_id(1)

    @pl.when(k == 0)
    def _():
        wb_ref[...] = w_ref[...].astype(jnp.bfloat16)

    xb = x_ref[...].astype(jnp.bfloat16)
    acc = jnp.dot(xb, wb_ref[...], preferred_element_type=jnp.float32)
    nc = o_ref.shape[1]
    o_ref[...] = acc[:, :nc] + b_ref[..., :nc]


def kernel(x, wp, bp):
    B, D = x.shape
    D_pad, C_pad = wp.shape
    C = _NUM_CLASSES

    tm = 512
    nb_inner = B // tm // 2
    grid = (2, nb_inner)

    cost = pl.CostEstimate(
        flops=2 * B * D_pad * C_pad,
        transcendentals=0,
        bytes_accessed=B * D * 4 + 2 * D_pad * C_pad * 4 + B * C * 4,
    )

    out = pl.pallas_call(
        _linear_body,
        out_shape=jax.ShapeDtypeStruct((B, C), jnp.float32),
        grid=grid,
        in_specs=[
            pl.BlockSpec((tm, D), lambda c, k: (c * nb_inner + k, 0)),
            pl.BlockSpec((D_pad, C_pad), lambda c, k: (0, 0)),
            pl.BlockSpec((1, C_pad), lambda c, k: (0, 0)),
        ],
        out_specs=pl.BlockSpec((tm, C), lambda c, k: (c * nb_inner + k, 0)),
        scratch_shapes=[pltpu.VMEM((D_pad, C_pad), jnp.bfloat16)],
        compiler_params=pltpu.CompilerParams(
            dimension_semantics=("parallel", "arbitrary"),
            vmem_limit_bytes=56 * 1024 * 1024,
        ),
        cost_estimate=cost,
    )(x, wp, bp)
    return out


# speedup vs baseline: 1.4081x; 1.0097x over previous
import jax
import jax.numpy as jnp
from jax.experimental import pallas as pl
from jax.experimental.pallas import tpu as pltpu

_NUM_CLASSES = 1000


def _linear_body(x_ref, w_ref, b_ref, o_ref, wb_ref):
    k = pl.program_id(1)

    @pl.when(k == 0)
    def _():
        wb_ref[...] = w_ref[...].astype(jnp.bfloat16)

    xb = x_ref[...].astype(jnp.bfloat16)
    acc = jnp.dot(xb, wb_ref[...], preferred_element_type=jnp.float32)
    nc = o_ref.shape[1]
    o_ref[...] = acc[:, :nc] + b_ref[..., :nc]


def kernel(x, wp, bp):
    B, D = x.shape
    D_pad, C_pad = wp.shape
    C = _NUM_CLASSES

    tm = 1024
    nb_inner = B // tm // 2
    grid = (2, nb_inner)

    cost = pl.CostEstimate(
        flops=2 * B * D_pad * C_pad,
        transcendentals=0,
        bytes_accessed=B * D * 4 + 2 * D_pad * C_pad * 4 + B * C * 4,
    )

    out = pl.pallas_call(
        _linear_body,
        out_shape=jax.ShapeDtypeStruct((B, C), jnp.float32),
        grid=grid,
        in_specs=[
            pl.BlockSpec((tm, D), lambda c, k: (c * nb_inner + k, 0)),
            pl.BlockSpec((D_pad, C_pad), lambda c, k: (0, 0)),
            pl.BlockSpec((1, C_pad), lambda c, k: (0, 0)),
        ],
        out_specs=pl.BlockSpec((tm, C), lambda c, k: (c * nb_inner + k, 0)),
        scratch_shapes=[pltpu.VMEM((D_pad, C_pad), jnp.bfloat16)],
        compiler_params=pltpu.CompilerParams(
            dimension_semantics=("parallel", "arbitrary"),
            vmem_limit_bytes=56 * 1024 * 1024,
        ),
        cost_estimate=cost,
    )(x, wp, bp)
    return out
